# Initial kernel scaffold; baseline (speedup 1.0000x reference)
import jax
import jax.numpy as jnp
from jax.experimental import pallas as pl


def kernel(token_ids, atom_embeddings, log_weight_embeddings):
    raise NotImplementedError("write your pallas kernel here")


# speedup vs baseline: 1.1827x; 1.1827x over previous
import functools

import jax
import jax.numpy as jnp
from jax import lax
from jax.experimental import pallas as pl
from jax.experimental.pallas import tpu as pltpu
from jax.experimental.pallas import tpu_sc as plsc

_D = 512
_NA = 8
_CH = 64


@functools.partial(jax.jit, static_argnums=())
def _lookup(ids, atom_embeddings, log_weight_embeddings):
    B = ids.shape[0]
    info = plsc.get_sparse_core_info()
    nc, ns = info.num_cores, info.num_subcores
    nw = nc * ns
    b_per_w = B // nw
    n_ch = b_per_w // _CH

    mesh = plsc.VectorSubcoreMesh(core_axis_name="c", subcore_axis_name="s")

    @functools.partial(
        pl.kernel,
        mesh=mesh,
        out_type=(
            jax.ShapeDtypeStruct((B, _D), jnp.float32),
            jax.ShapeDtypeStruct((B, _NA), jnp.float32),
        ),
        scratch_types=[
            pltpu.VMEM((b_per_w,), jnp.int32),
            pltpu.VMEM((b_per_w, _NA), jnp.float32),
            pltpu.VMEM((_CH, _D), jnp.float32),
            pltpu.VMEM((_CH, _D), jnp.float32),
            pltpu.SemaphoreType.DMA,
            pltpu.SemaphoreType.DMA,
            pltpu.SemaphoreType.DMA,
        ],
    )
    def k(ids_hbm, atoms_hbm, lw_hbm, out_a, out_w,
          idx_v, wrows_v, buf0, buf1, sem0, sem1, semw):
        wid = lax.axis_index("s") * nc + lax.axis_index("c")
        base = wid * b_per_w

        pltpu.sync_copy(ids_hbm.at[pl.ds(base, b_per_w)], idx_v)

        bufs = (buf0, buf1)
        sems = (sem0, sem1)

        cp = pltpu.async_copy(
            atoms_hbm.at[idx_v.at[pl.ds(0, _CH)]], bufs[0], sems[0])
        prev = (cp, 0, 0)

        def wbody(g, carry):
            vec = idx_v[pl.ds(g * 16, 16)]
            for j in range(16):
                t = vec[j]
                pltpu.async_copy(lw_hbm.at[pl.ds(t, 1)],
                                 wrows_v.at[pl.ds(g * 16 + j, 1)], semw)
            return carry
        lax.fori_loop(0, b_per_w // 16, wbody, 0)

        for c in range(1, n_ch):
            s = c & 1
            cp = pltpu.async_copy(
                atoms_hbm.at[idx_v.at[pl.ds(c * _CH, _CH)]], bufs[s], sems[s])
            pcp, ps, pc_i = prev
            pcp.wait()
            pltpu.sync_copy(bufs[ps],
                            out_a.at[pl.ds(base + pc_i * _CH, _CH)])
            prev = (cp, s, c)
        pcp, ps, pc_i = prev
        pcp.wait()
        pltpu.sync_copy(bufs[ps], out_a.at[pl.ds(base + pc_i * _CH, _CH)])

        pltpu.make_async_copy(lw_hbm.at[pl.ds(0, b_per_w)], wrows_v, semw).wait()
        pltpu.sync_copy(wrows_v, out_w.at[pl.ds(base, b_per_w)])

    return k(ids, atom_embeddings, log_weight_embeddings)


def kernel(token_ids, atom_embeddings, log_weight_embeddings):
    Bt, S = token_ids.shape
    ids = token_ids.reshape(-1).astype(jnp.int32)
    atoms_flat, lw = _lookup(ids, atom_embeddings, log_weight_embeddings)
    atoms = atoms_flat.reshape(Bt, S, _NA, _D // _NA)
    log_weights = lw.reshape(Bt, S, _NA)
    return (atoms, log_weights)
